# Initial kernel scaffold; baseline (speedup 1.0000x reference)
import jax
import jax.numpy as jnp
from jax.experimental import pallas as pl


def kernel(x, edge_embedding, edge_paths, edge_vector):
    raise NotImplementedError("write your pallas kernel here")


# speedup vs baseline: 53.8316x; 53.8316x over previous
import functools

import jax
import jax.numpy as jnp
from jax import lax
from jax.experimental import pallas as pl
from jax.experimental.pallas import tpu as pltpu
from jax.experimental.pallas import tpu_sc as plsc

N = 256
E = 8192
D = 64
P = 5
LANES = 16

NUM_PAIRS = N * N
NW = 32
PAIRS_PER_W = NUM_PAIRS // NW
GROUPS = PAIRS_PER_W // LANES


def _wt_body(ev_ref, ee_ref, out_ref):
    out_ref[...] = lax.dot_general(
        ev_ref[...],
        ee_ref[...],
        dimension_numbers=(((1,), (1,)), ((), ())),
        preferred_element_type=jnp.float32,
        precision=lax.Precision.HIGHEST,
    )


def _compute_wt(ev_pad, ee):
    return pl.pallas_call(
        _wt_body,
        out_shape=jax.ShapeDtypeStruct((8, E), jnp.float32),
    )(ev_pad, ee)


_sc_mesh = plsc.VectorSubcoreMesh(core_axis_name="c", subcore_axis_name="s")


@functools.partial(
    pl.kernel,
    out_type=jax.ShapeDtypeStruct((NUM_PAIRS,), jnp.float32),
    mesh=_sc_mesh,
    compiler_params=pltpu.CompilerParams(needs_layout_passes=False),
    scratch_types=[
        pltpu.VMEM((P * E,), jnp.float32),
        pltpu.VMEM((P * PAIRS_PER_W,), jnp.int32),
        pltpu.VMEM((PAIRS_PER_W,), jnp.float32),
    ],
)
def _sc_encode(wt_hbm, idx_hbm, out_hbm, table_v, idx_v, out_v):
    wid = lax.axis_index("s") * 2 + lax.axis_index("c")
    base = wid * PAIRS_PER_W
    pltpu.sync_copy(wt_hbm, table_v)
    pltpu.sync_copy(idx_hbm.at[wid], idx_v)

    def body(g, carry):
        off = g * LANES
        acc = jnp.zeros((LANES,), jnp.float32)
        cnt = jnp.zeros((LANES,), jnp.float32)
        for p in range(P):
            idx = idx_v[pl.ds(p * PAIRS_PER_W + off, LANES)]
            mask = idx >= 0
            safe = jnp.maximum(idx, 0) + (p * E)
            val = plsc.load_gather(table_v, [safe])
            acc = acc + jnp.where(mask, val, 0.0)
            cnt = cnt + jnp.where(mask, 1.0, 0.0)
        out_v[pl.ds(off, LANES)] = acc / jnp.maximum(cnt, 1.0)
        return carry

    lax.fori_loop(0, GROUPS, body, 0)
    pltpu.sync_copy(out_v, out_hbm.at[pl.ds(base, PAIRS_PER_W)])


def kernel(x, edge_embedding, edge_paths, edge_vector):
    del x
    ev_pad = jnp.zeros((8, D), jnp.float32).at[:P].set(edge_vector)
    wt = _compute_wt(ev_pad, edge_embedding)[:P].reshape(-1)
    idx = edge_paths.astype(jnp.int32)
    idx3 = idx.reshape(NW, PAIRS_PER_W, P).transpose(0, 2, 1).reshape(NW, -1)
    out_flat = _sc_encode(wt, idx3)
    return out_flat.reshape(N, N)


# speedup vs baseline: 59.5287x; 1.1058x over previous
import functools

import jax
import jax.numpy as jnp
from jax import lax
from jax.experimental import pallas as pl
from jax.experimental.pallas import tpu as pltpu
from jax.experimental.pallas import tpu_sc as plsc

N = 256
E = 8192
D = 64
P = 5
LANES = 16

NUM_PAIRS = N * N
NW = 32
PAIRS_PER_W = NUM_PAIRS // NW
GROUPS = PAIRS_PER_W // LANES


def _wt_body(ev_ref, ee_ref, out_ref):
    out_ref[...] = lax.dot_general(
        ev_ref[...],
        ee_ref[...],
        dimension_numbers=(((1,), (1,)), ((), ())),
        preferred_element_type=jnp.float32,
        precision=lax.Precision.HIGHEST,
    )


def _compute_wt(ev, ee):
    return pl.pallas_call(
        _wt_body,
        out_shape=jax.ShapeDtypeStruct((P, E), jnp.float32),
    )(ev, ee)


_sc_mesh = plsc.VectorSubcoreMesh(core_axis_name="c", subcore_axis_name="s")


ROWS_PER_W = N // NW


@functools.partial(
    pl.kernel,
    out_type=jax.ShapeDtypeStruct((N, N), jnp.float32),
    mesh=_sc_mesh,
    compiler_params=pltpu.CompilerParams(needs_layout_passes=False),
    scratch_types=[
        pltpu.VMEM((P, E), jnp.float32),
        pltpu.VMEM((P, ROWS_PER_W, N), jnp.int32),
        pltpu.VMEM((ROWS_PER_W, N), jnp.float32),
        pltpu.SemaphoreType.DMA,
        pltpu.SemaphoreType.DMA,
    ],
)
def _sc_encode(wt_hbm, idx_hbm, out_hbm, table_v, idx_v, out_v, sem_t, sem_i):
    wid = lax.axis_index("s") * 2 + lax.axis_index("c")
    row0 = wid * ROWS_PER_W
    cp_t = pltpu.async_copy(wt_hbm, table_v, sem_t)
    cps = [pltpu.async_copy(idx_hbm.at[p, pl.ds(row0, ROWS_PER_W)], idx_v.at[p], sem_i)
           for p in range(P)]
    for cp in cps:
        cp.wait()
    cp_t.wait()

    out_cps = []
    for r in range(ROWS_PER_W):
        @plsc.parallel_loop(0, N // LANES, unroll=2)
        def body(g):
            off = g * LANES
            acc = jnp.zeros((LANES,), jnp.float32)
            cnt = jnp.zeros((LANES,), jnp.float32)
            for p in range(P):
                idx = idx_v[p, r, pl.ds(off, LANES)]
                mask = idx >= 0
                safe = jnp.maximum(idx, 0)
                val = plsc.load_gather(table_v, [jnp.full((LANES,), p, jnp.int32), safe])
                acc = acc + jnp.where(mask, val, 0.0)
                cnt = cnt + jnp.where(mask, 1.0, 0.0)
            out_v[r, pl.ds(off, LANES)] = acc / jnp.maximum(cnt, 1.0)

        out_cps.append(pltpu.async_copy(out_v.at[r], out_hbm.at[row0 + r], sem_t))
    for cp in out_cps:
        cp.wait()


def kernel(x, edge_embedding, edge_paths, edge_vector):
    del x
    wt = _compute_wt(edge_vector, edge_embedding)
    idx_t = edge_paths.astype(jnp.int32).transpose(2, 0, 1)
    return _sc_encode(wt, idx_t)


# speedup vs baseline: 65.0337x; 1.0925x over previous
import functools

import jax
import jax.numpy as jnp
from jax import lax
from jax.experimental import pallas as pl
from jax.experimental.pallas import tpu as pltpu
from jax.experimental.pallas import tpu_sc as plsc

N = 256
E = 8192
D = 64
P = 5
LANES = 16

NUM_PAIRS = N * N
NW = 32
PAIRS_PER_W = NUM_PAIRS // NW
GROUPS = PAIRS_PER_W // LANES


def _wt_body(ev_ref, ee_ref, out_ref):
    out_ref[...] = lax.dot_general(
        ev_ref[...],
        ee_ref[...],
        dimension_numbers=(((1,), (1,)), ((), ())),
        preferred_element_type=jnp.float32,
    )


def _compute_wt(ev_pad, ee):
    return pl.pallas_call(
        _wt_body,
        out_shape=jax.ShapeDtypeStruct((8, E), jnp.float32),
    )(ev_pad, ee)


_sc_mesh = plsc.VectorSubcoreMesh(core_axis_name="c", subcore_axis_name="s")


ROWS_PER_W = N // NW


@functools.partial(
    pl.kernel,
    out_type=jax.ShapeDtypeStruct((N, N), jnp.float32),
    mesh=_sc_mesh,
    compiler_params=pltpu.CompilerParams(needs_layout_passes=False),
    scratch_types=[
        pltpu.VMEM((P * E,), jnp.float32),
        pltpu.VMEM((P, ROWS_PER_W, N), jnp.int32),
        pltpu.VMEM((ROWS_PER_W, N), jnp.float32),
        pltpu.SemaphoreType.DMA,
        pltpu.SemaphoreType.DMA,
    ],
)
def _sc_encode(wt_hbm, idx_hbm, out_hbm, table_v, idx_v, out_v, sem_t, sem_i):
    wid = lax.axis_index("s") * 2 + lax.axis_index("c")
    row0 = wid * ROWS_PER_W
    cp_t = pltpu.async_copy(wt_hbm, table_v, sem_t)
    cps = [pltpu.async_copy(idx_hbm.at[p, pl.ds(row0, ROWS_PER_W)], idx_v.at[p], sem_i)
           for p in range(P)]
    for cp in cps:
        cp.wait()
    cp_t.wait()

    for r in range(ROWS_PER_W):
        @plsc.parallel_loop(0, N // LANES, unroll=2)
        def body(g):
            off = g * LANES
            acc = jnp.zeros((LANES,), jnp.float32)
            cnt = jnp.zeros((LANES,), jnp.float32)
            for p in range(P):
                idx = idx_v[p, r, pl.ds(off, LANES)]
                mask = idx >= 0
                safe = jnp.maximum(idx, 0) + (p * E)
                val = plsc.load_gather(table_v, [safe])
                acc = acc + jnp.where(mask, val, 0.0)
                cnt = cnt + jnp.where(mask, 1.0, 0.0)
            out_v[r, pl.ds(off, LANES)] = acc / jnp.maximum(cnt, 1.0)

    pltpu.sync_copy(out_v, out_hbm.at[pl.ds(row0, ROWS_PER_W)])


def kernel(x, edge_embedding, edge_paths, edge_vector):
    del x
    ev_pad = jnp.zeros((8, D), jnp.float32).at[:P].set(edge_vector)
    wt = _compute_wt(ev_pad, edge_embedding)[:P].reshape(-1)
    idx_t = edge_paths.astype(jnp.int32).transpose(2, 0, 1)
    return _sc_encode(wt, idx_t)


# speedup vs baseline: 67.7363x; 1.0416x over previous
import functools

import jax
import jax.numpy as jnp
from jax import lax
from jax.experimental import pallas as pl
from jax.experimental.pallas import tpu as pltpu
from jax.experimental.pallas import tpu_sc as plsc

N = 256
E = 8192
D = 64
P = 5
LANES = 16

NUM_PAIRS = N * N
NW = 32
PAIRS_PER_W = NUM_PAIRS // NW
GROUPS = PAIRS_PER_W // LANES


def _wt_body(ev_ref, ee_ref, out_ref):
    w = lax.dot_general(
        ev_ref[...],
        ee_ref[...],
        dimension_numbers=(((1,), (1,)), ((), ())),
        preferred_element_type=jnp.float32,
    )

    def pack(a, b):
        ha = lax.bitcast_convert_type(a.astype(jnp.bfloat16), jnp.uint16)
        hb = lax.bitcast_convert_type(b.astype(jnp.bfloat16), jnp.uint16)
        return (ha.astype(jnp.uint32) | (hb.astype(jnp.uint32) << 16)).astype(jnp.int32)

    out_ref[0:1, :] = pack(w[0:1], w[1:2])
    out_ref[1:2, :] = pack(w[2:3], w[3:4])
    out_ref[2:3, :] = lax.bitcast_convert_type(w[4:5], jnp.int32)


def _compute_wt(ev_pad, ee):
    return pl.pallas_call(
        _wt_body,
        out_shape=jax.ShapeDtypeStruct((3, E), jnp.int32),
    )(ev_pad, ee)


_sc_mesh = plsc.VectorSubcoreMesh(core_axis_name="c", subcore_axis_name="s")


ROWS_PER_W = N // NW


@functools.partial(
    pl.kernel,
    out_type=jax.ShapeDtypeStruct((N, N), jnp.float32),
    mesh=_sc_mesh,
    compiler_params=pltpu.CompilerParams(needs_layout_passes=False),
    scratch_types=[
        pltpu.VMEM((3 * E,), jnp.int32),
        pltpu.VMEM((P, ROWS_PER_W, N), jnp.int32),
        pltpu.VMEM((ROWS_PER_W, N), jnp.float32),
        pltpu.SemaphoreType.DMA,
        pltpu.SemaphoreType.DMA,
    ],
)
def _sc_encode(wt_hbm, idx_hbm, out_hbm, table_v, idx_v, out_v, sem_t, sem_i):
    wid = lax.axis_index("s") * 2 + lax.axis_index("c")
    row0 = wid * ROWS_PER_W
    cp_t = pltpu.async_copy(wt_hbm, table_v, sem_t)
    cps = [pltpu.async_copy(idx_hbm.at[p, pl.ds(row0, ROWS_PER_W)], idx_v.at[p], sem_i)
           for p in range(P)]
    for cp in cps:
        cp.wait()
    cp_t.wait()

    himask = jnp.full((LANES,), -65536, jnp.int32)
    for r in range(ROWS_PER_W):
        @plsc.parallel_loop(0, N // LANES, unroll=2)
        def body(g):
            off = g * LANES
            acc = jnp.zeros((LANES,), jnp.float32)
            cnt = jnp.zeros((LANES,), jnp.float32)
            idxs, masks = [], []
            for p in range(P):
                idx = idx_v[p, r, pl.ds(off, LANES)]
                idxs.append(jnp.maximum(idx, 0))
                masks.append(idx >= 0)
                cnt = cnt + jnp.where(idx >= 0, 1.0, 0.0)
            for q in range(2):
                v01 = plsc.load_gather(table_v, [idxs[2 * q] + q * E])
                v01b = plsc.load_gather(table_v, [idxs[2 * q + 1] + q * E])
                lo = plsc.bitcast(lax.shift_left(v01, 16), jnp.float32)
                hi = plsc.bitcast(jnp.bitwise_and(v01b, himask), jnp.float32)
                acc = acc + jnp.where(masks[2 * q], lo, 0.0)
                acc = acc + jnp.where(masks[2 * q + 1], hi, 0.0)
            v4 = plsc.load_gather(table_v, [idxs[4] + 2 * E])
            acc = acc + jnp.where(masks[4], plsc.bitcast(v4, jnp.float32), 0.0)
            out_v[r, pl.ds(off, LANES)] = acc / jnp.maximum(cnt, 1.0)

    pltpu.sync_copy(out_v, out_hbm.at[pl.ds(row0, ROWS_PER_W)])


def kernel(x, edge_embedding, edge_paths, edge_vector):
    del x
    ev_pad = jnp.zeros((8, D), jnp.float32).at[:P].set(edge_vector)
    wt = _compute_wt(ev_pad, edge_embedding).reshape(-1)
    idx_t = edge_paths.astype(jnp.int32).transpose(2, 0, 1)
    return _sc_encode(wt, idx_t)
